# Initial kernel scaffold; baseline (speedup 1.0000x reference)
import jax
import jax.numpy as jnp
from jax.experimental import pallas as pl


def kernel(z, pos_edge_index, neg_edge_index):
    raise NotImplementedError("write your pallas kernel here")


# speedup vs baseline: 3.5385x; 3.5385x over previous
import functools

import jax
import jax.numpy as jnp
from jax import lax
from jax.experimental import pallas as pl
from jax.experimental.pallas import tpu as pltpu
from jax.experimental.pallas import tpu_sc as plsc

N_NODES = 10000
D_FEAT = 128
E_POS = 160000
E_TOTAL = 320000

NC = 2
NS = 16
NW = NC * NS
EPW = E_TOTAL // NW
CHUNK = 80
NCHUNK = EPW // CHUNK


def _make_sc_logits():
    mesh = plsc.VectorSubcoreMesh(
        core_axis_name="c", subcore_axis_name="s", num_cores=NC, num_subcores=NS
    )

    @functools.partial(
        pl.kernel,
        mesh=mesh,
        out_type=jax.ShapeDtypeStruct((NW, NCHUNK, CHUNK), jnp.float32),
        scratch_types=[
            pltpu.VMEM((NCHUNK, CHUNK), jnp.int32),
            pltpu.VMEM((NCHUNK, CHUNK), jnp.int32),
            pltpu.VMEM((CHUNK, D_FEAT), jnp.float32),
            pltpu.VMEM((CHUNK, D_FEAT), jnp.float32),
            pltpu.VMEM((NCHUNK, CHUNK), jnp.float32),
            pltpu.SemaphoreType.DMA,
        ],
    )
    def sc_logits(z_hbm, src_hbm, dst_hbm, out_hbm,
                  idx_s, idx_d, rows_s, rows_d, outbuf, sem):
        wid = lax.axis_index("s") * NC + lax.axis_index("c")
        pltpu.sync_copy(src_hbm.at[wid], idx_s)
        pltpu.sync_copy(dst_hbm.at[wid], idx_d)

        lanes = lax.broadcasted_iota(jnp.int32, (16,), 0)
        dnums = lax.GatherDimensionNumbers(
            offset_dims=(), collapsed_slice_dims=(0,), start_index_map=(0,))

        def lane_shuffle(v, perm):
            return lax.gather(
                v, perm[:, None], dnums, slice_sizes=(1,),
                mode=lax.GatherScatterMode.PROMISE_IN_BOUNDS)

        def chunk_body(c, carry):
            cp_s = pltpu.async_copy(z_hbm.at[idx_s.at[c]], rows_s, sem)
            cp_d = pltpu.async_copy(z_hbm.at[idx_d.at[c]], rows_d, sem)
            cp_s.wait()
            cp_d.wait()

            def group_body(g, carry2):
                lvec = jnp.zeros((16,), jnp.float32)
                for j in range(16):
                    e = g * 16 + j
                    acc = rows_s[e, pl.ds(0, 16)] * rows_d[e, pl.ds(0, 16)]
                    for k in range(1, D_FEAT // 16):
                        acc = acc + (rows_s[e, pl.ds(k * 16, 16)]
                                     * rows_d[e, pl.ds(k * 16, 16)])
                    for k in range(4):
                        acc = acc + lane_shuffle(acc, lanes ^ (1 << k))
                    lvec = jnp.where(lanes == j, acc, lvec)
                outbuf[c, pl.ds(g * 16, 16)] = lvec
                return carry2

            lax.fori_loop(0, CHUNK // 16, group_body, 0)
            return carry

        lax.fori_loop(0, NCHUNK, chunk_body, 0)
        pltpu.sync_copy(outbuf, out_hbm.at[wid])

    return sc_logits


def _tc_loss_body(l_ref, o_ref):
    l = l_ref[...]
    rows = lax.broadcasted_iota(jnp.int32, l.shape, 0)
    x = jnp.where(rows < E_POS // 128, -l, l)
    sp = jnp.maximum(x, 0.0) + jnp.log1p(jnp.exp(-jnp.abs(x)))
    o_ref[0, 0] = jnp.sum(sp) * (1.0 / E_TOTAL)


_tc_loss = pl.pallas_call(
    _tc_loss_body,
    out_shape=jax.ShapeDtypeStruct((1, 1), jnp.float32),
    out_specs=pl.BlockSpec(memory_space=pltpu.SMEM),
)


@jax.jit
def kernel(z, pos_edge_index, neg_edge_index):
    total = jnp.concatenate([pos_edge_index, neg_edge_index], axis=1)
    src = total[0].reshape(NW, NCHUNK, CHUNK)
    dst = total[1].reshape(NW, NCHUNK, CHUNK)
    logits = _make_sc_logits()(z, src, dst)
    loss = _tc_loss(logits.reshape(E_TOTAL // 128, 128))
    return loss[0, 0]


# speedup vs baseline: 4.9252x; 1.3919x over previous
import functools

import jax
import jax.numpy as jnp
from jax import lax
from jax.experimental import pallas as pl
from jax.experimental.pallas import tpu as pltpu
from jax.experimental.pallas import tpu_sc as plsc

N_NODES = 10000
D_FEAT = 128
E_POS = 160000
E_TOTAL = 320000

NC = 2
NS = 16
NW = NC * NS
EPW = E_TOTAL // NW
CHUNK = 80
NCHUNK = EPW // CHUNK


def _make_sc_logits():
    mesh = plsc.VectorSubcoreMesh(
        core_axis_name="c", subcore_axis_name="s", num_cores=NC, num_subcores=NS
    )

    @functools.partial(
        pl.kernel,
        mesh=mesh,
        out_type=jax.ShapeDtypeStruct((NW, NCHUNK, CHUNK), jnp.float32),
        scratch_types=[
            pltpu.VMEM((NCHUNK, CHUNK), jnp.int32),
            pltpu.VMEM((NCHUNK, CHUNK), jnp.int32),
            pltpu.VMEM((CHUNK, D_FEAT), jnp.float32),
            pltpu.VMEM((CHUNK, D_FEAT), jnp.float32),
            pltpu.VMEM((CHUNK, D_FEAT), jnp.float32),
            pltpu.VMEM((CHUNK, D_FEAT), jnp.float32),
            pltpu.VMEM((NCHUNK, CHUNK), jnp.float32),
            pltpu.SemaphoreType.DMA,
            pltpu.SemaphoreType.DMA,
        ],
    )
    def sc_logits(z_hbm, src_hbm, dst_hbm, out_hbm,
                  idx_s, idx_d, rows_s0, rows_d0, rows_s1, rows_d1,
                  outbuf, sem0, sem1):
        wid = lax.axis_index("s") * NC + lax.axis_index("c")
        pltpu.sync_copy(src_hbm.at[wid], idx_s)
        pltpu.sync_copy(dst_hbm.at[wid], idx_d)

        lanes = lax.broadcasted_iota(jnp.int32, (16,), 0)
        dnums = lax.GatherDimensionNumbers(
            offset_dims=(), collapsed_slice_dims=(0,), start_index_map=(0,))

        def lane_shuffle(v, perm):
            return lax.gather(
                v, perm[:, None], dnums, slice_sizes=(1,),
                mode=lax.GatherScatterMode.PROMISE_IN_BOUNDS)

        def start(c, rs, rd, sem):
            pltpu.async_copy(z_hbm.at[idx_s.at[c]], rs, sem)
            pltpu.async_copy(z_hbm.at[idx_d.at[c]], rd, sem)

        def drain(c, rs, rd, sem):
            pltpu.make_async_copy(z_hbm.at[idx_s.at[c]], rs, sem).wait()
            pltpu.make_async_copy(z_hbm.at[idx_d.at[c]], rd, sem).wait()

        def compute(c, rs, rd):
            def group_body(g, carry2):
                lvec = jnp.zeros((16,), jnp.float32)
                for j in range(16):
                    e = g * 16 + j
                    acc = rs[e, pl.ds(0, 16)] * rd[e, pl.ds(0, 16)]
                    for k in range(1, D_FEAT // 16):
                        acc = acc + (rs[e, pl.ds(k * 16, 16)]
                                     * rd[e, pl.ds(k * 16, 16)])
                    for k in range(4):
                        acc = acc + lane_shuffle(acc, lanes ^ (1 << k))
                    lvec = jnp.where(lanes == j, acc, lvec)
                outbuf[c, pl.ds(g * 16, 16)] = lvec
                return carry2

            lax.fori_loop(0, CHUNK // 16, group_body, 0)

        start(0, rows_s0, rows_d0, sem0)

        def pair_body(cc, carry):
            c0 = 2 * cc
            start(c0 + 1, rows_s1, rows_d1, sem1)
            drain(c0, rows_s0, rows_d0, sem0)
            compute(c0, rows_s0, rows_d0)
            start(c0 + 2, rows_s0, rows_d0, sem0)
            drain(c0 + 1, rows_s1, rows_d1, sem1)
            compute(c0 + 1, rows_s1, rows_d1)
            return carry

        lax.fori_loop(0, (NCHUNK - 1) // 2, pair_body, 0)
        drain(NCHUNK - 1, rows_s0, rows_d0, sem0)
        compute(NCHUNK - 1, rows_s0, rows_d0)
        pltpu.sync_copy(outbuf, out_hbm.at[wid])

    return sc_logits


def _tc_loss_body(l_ref, o_ref):
    l = l_ref[...]
    rows = lax.broadcasted_iota(jnp.int32, l.shape, 0)
    x = jnp.where(rows < E_POS // 128, -l, l)
    sp = jnp.maximum(x, 0.0) + jnp.log1p(jnp.exp(-jnp.abs(x)))
    o_ref[0, 0] = jnp.sum(sp) * (1.0 / E_TOTAL)


_tc_loss = pl.pallas_call(
    _tc_loss_body,
    out_shape=jax.ShapeDtypeStruct((1, 1), jnp.float32),
    out_specs=pl.BlockSpec(memory_space=pltpu.SMEM),
)


@jax.jit
def kernel(z, pos_edge_index, neg_edge_index):
    total = jnp.concatenate([pos_edge_index, neg_edge_index], axis=1)
    src = total[0].reshape(NW, NCHUNK, CHUNK)
    dst = total[1].reshape(NW, NCHUNK, CHUNK)
    logits = _make_sc_logits()(z, src, dst)
    loss = _tc_loss(logits.reshape(E_TOTAL // 128, 128))
    return loss[0, 0]


# speedup vs baseline: 7.5870x; 1.5404x over previous
import functools

import jax
import jax.numpy as jnp
from jax import lax
from jax.experimental import pallas as pl
from jax.experimental.pallas import tpu as pltpu
from jax.experimental.pallas import tpu_sc as plsc

N_NODES = 10000
D_FEAT = 128
E_POS = 160000
E_TOTAL = 320000

NC = 2
NS = 16
NW = NC * NS
EPW = E_TOTAL // NW
CHUNK = 80
NCHUNK = EPW // CHUNK


def _make_sc_logits():
    mesh = plsc.VectorSubcoreMesh(
        core_axis_name="c", subcore_axis_name="s", num_cores=NC, num_subcores=NS
    )

    @functools.partial(
        pl.kernel,
        mesh=mesh,
        compiler_params=pltpu.CompilerParams(
            needs_layout_passes=False, use_tc_tiling_on_sc=False),
        out_type=jax.ShapeDtypeStruct((NW, NCHUNK, CHUNK), jnp.float32),
        scratch_types=[
            pltpu.VMEM((NCHUNK, CHUNK), jnp.int32),
            pltpu.VMEM((NCHUNK, CHUNK), jnp.int32),
            pltpu.VMEM((CHUNK, D_FEAT // 2), jnp.int32),
            pltpu.VMEM((CHUNK, D_FEAT // 2), jnp.int32),
            pltpu.VMEM((CHUNK, D_FEAT // 2), jnp.int32),
            pltpu.VMEM((CHUNK, D_FEAT // 2), jnp.int32),
            pltpu.VMEM((NCHUNK, CHUNK), jnp.float32),
            pltpu.SemaphoreType.DMA,
            pltpu.SemaphoreType.DMA,
        ],
    )
    def sc_logits(z_hbm, src_hbm, dst_hbm, out_hbm,
                  idx_s, idx_d, rows_s0, rows_d0, rows_s1, rows_d1,
                  outbuf, sem0, sem1):
        wid = lax.axis_index("s") * NC + lax.axis_index("c")
        pltpu.sync_copy(src_hbm.at[wid], idx_s)
        pltpu.sync_copy(dst_hbm.at[wid], idx_d)

        lanes = lax.broadcasted_iota(jnp.int32, (16,), 0)
        dnums = lax.GatherDimensionNumbers(
            offset_dims=(), collapsed_slice_dims=(0,), start_index_map=(0,))

        def lane_shuffle(v, perm):
            return lax.gather(
                v, perm[:, None], dnums, slice_sizes=(1,),
                mode=lax.GatherScatterMode.PROMISE_IN_BOUNDS)

        def start(c, rs, rd, sem):
            pltpu.async_copy(z_hbm.at[idx_s.at[c]], rs, sem)
            pltpu.async_copy(z_hbm.at[idx_d.at[c]], rd, sem)

        def drain(c, rs, rd, sem):
            pltpu.make_async_copy(z_hbm.at[idx_s.at[c]], rs, sem).wait()
            pltpu.make_async_copy(z_hbm.at[idx_d.at[c]], rd, sem).wait()

        def compute(c, rs, rd):
            def group_body(g, carry2):
                lvec = jnp.zeros((16,), jnp.float32)
                for j in range(16):
                    e = g * 16 + j
                    acc = jnp.zeros((16,), jnp.float32)
                    for k in range(D_FEAT // 32):
                        ws = rs[e, pl.ds(k * 16, 16)]
                        wd = rd[e, pl.ds(k * 16, 16)]
                        slo = plsc.bitcast(lax.shift_left(ws, 16), jnp.float32)
                        shi = plsc.bitcast(ws & jnp.int32(-65536), jnp.float32)
                        dlo = plsc.bitcast(lax.shift_left(wd, 16), jnp.float32)
                        dhi = plsc.bitcast(wd & jnp.int32(-65536), jnp.float32)
                        acc = acc + slo * dlo + shi * dhi
                    for k in range(4):
                        acc = acc + lane_shuffle(acc, lanes ^ (1 << k))
                    lvec = jnp.where(lanes == j, acc, lvec)
                outbuf[c, pl.ds(g * 16, 16)] = lvec
                return carry2

            lax.fori_loop(0, CHUNK // 16, group_body, 0)

        start(0, rows_s0, rows_d0, sem0)

        def pair_body(cc, carry):
            c0 = 2 * cc
            start(c0 + 1, rows_s1, rows_d1, sem1)
            drain(c0, rows_s0, rows_d0, sem0)
            compute(c0, rows_s0, rows_d0)
            start(c0 + 2, rows_s0, rows_d0, sem0)
            drain(c0 + 1, rows_s1, rows_d1, sem1)
            compute(c0 + 1, rows_s1, rows_d1)
            return carry

        lax.fori_loop(0, (NCHUNK - 1) // 2, pair_body, 0)
        drain(NCHUNK - 1, rows_s0, rows_d0, sem0)
        compute(NCHUNK - 1, rows_s0, rows_d0)
        pltpu.sync_copy(outbuf, out_hbm.at[wid])

    return sc_logits


def _tc_loss_body(l_ref, o_ref):
    l = l_ref[...]
    rows = lax.broadcasted_iota(jnp.int32, l.shape, 0)
    x = jnp.where(rows < E_POS // 128, -l, l)
    sp = jnp.maximum(x, 0.0) + jnp.log1p(jnp.exp(-jnp.abs(x)))
    o_ref[0, 0] = jnp.sum(sp) * (1.0 / E_TOTAL)


_tc_loss = pl.pallas_call(
    _tc_loss_body,
    out_shape=jax.ShapeDtypeStruct((1, 1), jnp.float32),
    out_specs=pl.BlockSpec(memory_space=pltpu.SMEM),
)


@jax.jit
def kernel(z, pos_edge_index, neg_edge_index):
    total = jnp.concatenate([pos_edge_index, neg_edge_index], axis=1)
    src = total[0].reshape(NW, NCHUNK, CHUNK)
    dst = total[1].reshape(NW, NCHUNK, CHUNK)
    z_packed = lax.bitcast_convert_type(
        z.astype(jnp.bfloat16).reshape(N_NODES, D_FEAT // 2, 2), jnp.int32)
    logits = _make_sc_logits()(z_packed, src, dst)
    loss = _tc_loss(logits.reshape(E_TOTAL // 128, 128))
    return loss[0, 0]


# speedup vs baseline: 11.4910x; 1.5146x over previous
import functools

import jax
import jax.numpy as jnp
from jax import lax
from jax.experimental import pallas as pl
from jax.experimental.pallas import tpu as pltpu
from jax.experimental.pallas import tpu_sc as plsc

N_NODES = 10000
D_FEAT = 128
E_POS = 160000
E_TOTAL = 320000

NC = 2
NS = 16
NW = NC * NS
EPW = E_TOTAL // NW
CHUNK = 80
NCHUNK = EPW // CHUNK


def _make_sc_logits():
    mesh = plsc.VectorSubcoreMesh(
        core_axis_name="c", subcore_axis_name="s", num_cores=NC, num_subcores=NS
    )

    @functools.partial(
        pl.kernel,
        mesh=mesh,
        compiler_params=pltpu.CompilerParams(
            needs_layout_passes=False, use_tc_tiling_on_sc=False),
        out_type=jax.ShapeDtypeStruct((NW, NCHUNK, CHUNK), jnp.float32),
        scratch_types=[
            pltpu.VMEM((NCHUNK, CHUNK), jnp.int32),
            pltpu.VMEM((NCHUNK, CHUNK), jnp.int32),
            pltpu.VMEM((CHUNK, D_FEAT), jnp.bfloat16),
            pltpu.VMEM((CHUNK, D_FEAT), jnp.bfloat16),
            pltpu.VMEM((CHUNK, D_FEAT), jnp.bfloat16),
            pltpu.VMEM((CHUNK, D_FEAT), jnp.bfloat16),
            pltpu.VMEM((NCHUNK, CHUNK), jnp.float32),
            pltpu.SemaphoreType.DMA,
            pltpu.SemaphoreType.DMA,
        ],
    )
    def sc_logits(z_hbm, src_hbm, dst_hbm, out_hbm,
                  idx_s, idx_d, rows_s0, rows_d0, rows_s1, rows_d1,
                  outbuf, sem0, sem1):
        wid = lax.axis_index("s") * NC + lax.axis_index("c")
        pltpu.sync_copy(src_hbm.at[wid], idx_s)
        pltpu.sync_copy(dst_hbm.at[wid], idx_d)

        lanes = lax.broadcasted_iota(jnp.int32, (16,), 0)
        dnums = lax.GatherDimensionNumbers(
            offset_dims=(), collapsed_slice_dims=(0,), start_index_map=(0,))

        def lane_shuffle(v, perm):
            return lax.gather(
                v, perm[:, None], dnums, slice_sizes=(1,),
                mode=lax.GatherScatterMode.PROMISE_IN_BOUNDS)

        def start(c, rs, rd, sem):
            pltpu.async_copy(z_hbm.at[idx_s.at[c]], rs, sem)
            pltpu.async_copy(z_hbm.at[idx_d.at[c]], rd, sem)

        def drain(c, rs, rd, sem):
            pltpu.make_async_copy(z_hbm.at[idx_s.at[c]], rs, sem).wait()
            pltpu.make_async_copy(z_hbm.at[idx_d.at[c]], rd, sem).wait()

        def compute(c, rs, rd):
            def group_body(g, carry2):
                lvec = jnp.zeros((16,), jnp.float32)
                for j in range(16):
                    e = g * 16 + j
                    accb = rs[e, pl.ds(0, 32)] * rd[e, pl.ds(0, 32)]
                    for k in range(1, D_FEAT // 32):
                        accb = accb + (rs[e, pl.ds(k * 32, 32)]
                                       * rd[e, pl.ds(k * 32, 32)])
                    a0, a1 = plsc.unpack(
                        accb, format=plsc.PackFormat.INTERLEAVED)
                    acc = jnp.sum(a0 + a1)
                    lvec = jnp.where(lanes == j, acc, lvec)
                outbuf[c, pl.ds(g * 16, 16)] = lvec
                return carry2

            lax.fori_loop(0, CHUNK // 16, group_body, 0)

        start(0, rows_s0, rows_d0, sem0)

        def pair_body(cc, carry):
            c0 = 2 * cc
            start(c0 + 1, rows_s1, rows_d1, sem1)
            drain(c0, rows_s0, rows_d0, sem0)
            compute(c0, rows_s0, rows_d0)
            start(c0 + 2, rows_s0, rows_d0, sem0)
            drain(c0 + 1, rows_s1, rows_d1, sem1)
            compute(c0 + 1, rows_s1, rows_d1)
            return carry

        lax.fori_loop(0, (NCHUNK - 1) // 2, pair_body, 0)
        drain(NCHUNK - 1, rows_s0, rows_d0, sem0)
        compute(NCHUNK - 1, rows_s0, rows_d0)
        pltpu.sync_copy(outbuf, out_hbm.at[wid])

    return sc_logits


def _tc_loss_body(l_ref, o_ref):
    l = l_ref[...]
    rows = lax.broadcasted_iota(jnp.int32, l.shape, 0)
    x = jnp.where(rows < E_POS // 128, -l, l)
    sp = jnp.maximum(x, 0.0) + jnp.log1p(jnp.exp(-jnp.abs(x)))
    o_ref[0, 0] = jnp.sum(sp) * (1.0 / E_TOTAL)


_tc_loss = pl.pallas_call(
    _tc_loss_body,
    out_shape=jax.ShapeDtypeStruct((1, 1), jnp.float32),
    out_specs=pl.BlockSpec(memory_space=pltpu.SMEM),
)


@jax.jit
def kernel(z, pos_edge_index, neg_edge_index):
    total = jnp.concatenate([pos_edge_index, neg_edge_index], axis=1)
    src = total[0].reshape(NW, NCHUNK, CHUNK)
    dst = total[1].reshape(NW, NCHUNK, CHUNK)
    logits = _make_sc_logits()(z.astype(jnp.bfloat16), src, dst)
    loss = _tc_loss(logits.reshape(E_TOTAL // 128, 128))
    return loss[0, 0]
